# baseline (speedup 1.0000x reference)
import functools

import jax
import jax.numpy as jnp
from jax import lax
from jax.experimental import pallas as pl
from jax.experimental.pallas import tpu as pltpu
from jax.experimental.pallas import tpu_sc as plsc

N = 10000
E = 320000
D_IN = 128
D_H = 64
G = 64

NC = 2
NS = 16
CH = 128
NW = NC * NS
K_T = 80
NP = 2
PASS = K_T // NP
R_CHUNKS = NW * K_T
E_PAD = R_CHUNKS * CH
ROWS_T = 640
A_ROWS = NS * ROWS_T

NBUF = 4
LA = NBUF // 2
K_DEG = R_CHUNKS // NW
CVT_U = 8
D_P = D_H // 2


def _sc_agg_body(hp_hbm, src_hbm, dst_hbm, zeros_hbm, out_hbm,
                 acc, src_idx, dst_idx, rows16, rows, *sems):
    c = lax.axis_index("c")
    s = lax.axis_index("s")
    wid = c * NS + s
    row0 = s * ROWS_T

    pltpu.sync_copy(zeros_hbm, rows.at[0])
    for r in range(ROWS_T // CH):
        pltpu.sync_copy(rows.at[0], acc.at[pl.ds(row0 + r * CH, CH)])

    plsc.subcore_barrier()

    gsem = sems[:NBUF]
    ssem = sems[NBUF:]

    hi_mask = jnp.int32(-65536)

    def _convert(b):
        def crow(r, carry):
            for u in range(CVT_U):
                rr = r * CVT_U + u
                w1 = rows16[b, rr, 0:16]
                w2 = rows16[b, rr, 16:32]
                rows[b, rr, 0:16] = plsc.bitcast(w1 << 16, jnp.float32)
                rows[b, rr, 16:32] = plsc.bitcast(w1 & hi_mask, jnp.float32)
                rows[b, rr, 32:48] = plsc.bitcast(w2 << 16, jnp.float32)
                rows[b, rr, 48:64] = plsc.bitcast(w2 & hi_mask, jnp.float32)
            return carry

        lax.fori_loop(0, CH // CVT_U, crow, 0)

    def _step(j, b, swait, gfire):
        pltpu.make_async_copy(
            hp_hbm.at[src_idx.at[j]], rows16.at[b], gsem[b]).wait()
        _convert(b)
        pltpu.async_copy(rows.at[b], acc.at[dst_idx.at[j]], ssem[b],
                         add=True)
        b2 = (b + LA) % NBUF
        if swait:
            pltpu.make_async_copy(
                rows.at[b2], acc.at[dst_idx.at[j - LA]], ssem[b2]).wait()
        if gfire:
            pltpu.async_copy(hp_hbm.at[src_idx.at[j + LA]], rows16.at[b2],
                             gsem[b2])

    for p in range(NP):
        start = wid * K_T + p * PASS
        pltpu.sync_copy(src_hbm.at[pl.ds(start, PASS)], src_idx)
        pltpu.sync_copy(dst_hbm.at[pl.ds(start, PASS)], dst_idx)

        for b in range(LA):
            pltpu.async_copy(hp_hbm.at[src_idx.at[b]], rows16.at[b], gsem[b])

        for b in range(NBUF):
            _step(b, b, b >= LA, True)

        def step(i, carry):
            for b in range(NBUF):
                _step(i * NBUF + b, b, True, True)
            return carry

        lax.fori_loop(1, PASS // NBUF - 1, step, 0)

        for b in range(NBUF):
            _step(PASS - NBUF + b, b, True, b < NBUF - LA)

        for b in range(NBUF - LA, NBUF):
            pltpu.make_async_copy(
                rows.at[b], acc.at[dst_idx.at[PASS - NBUF + b]],
                ssem[b]).wait()

    plsc.subcore_barrier()

    for r in range(ROWS_T // CH):
        pltpu.sync_copy(acc.at[pl.ds(row0 + r * CH, CH)], rows.at[0])
        pltpu.sync_copy(rows.at[0], out_hbm.at[c, pl.ds(row0 + r * CH, CH)])


def _sc_deg_body(dst_hbm, ones_hbm, zeros_hbm, out_hbm,
                 acc, dst_idx, rows):
    c = lax.axis_index("c")
    s = lax.axis_index("s")
    wid = c * NS + s
    row0 = s * ROWS_T

    pltpu.sync_copy(zeros_hbm, rows)
    for r in range(ROWS_T // CH):
        pltpu.sync_copy(rows, acc.at[pl.ds(row0 + r * CH, CH)])
    pltpu.sync_copy(dst_hbm.at[pl.ds(wid * K_DEG, K_DEG)], dst_idx)
    pltpu.sync_copy(ones_hbm, rows)

    plsc.subcore_barrier()

    def body(j, carry):
        pltpu.sync_copy(rows, acc.at[dst_idx.at[j]], add=True)
        return carry

    lax.fori_loop(0, K_DEG, body, 0)

    plsc.subcore_barrier()

    for r in range(ROWS_T // CH):
        pltpu.sync_copy(acc.at[pl.ds(row0 + r * CH, CH)], rows)
        pltpu.sync_copy(rows, out_hbm.at[c, pl.ds(row0 + r * CH, CH)])


@functools.lru_cache(maxsize=1)
def _sc_kernels():
    mesh = plsc.VectorSubcoreMesh(
        core_axis_name="c", subcore_axis_name="s",
        num_cores=NC, num_subcores=NS,
    )
    out_t = jax.ShapeDtypeStruct((NC, A_ROWS, D_H), jnp.float32)
    params = pltpu.CompilerParams(use_tc_tiling_on_sc=False,
                                  needs_layout_passes=False)
    agg = pl.kernel(
        _sc_agg_body,
        out_type=out_t,
        mesh=mesh,
        compiler_params=params,
        scratch_types=[
            pltpu.VMEM_SHARED((A_ROWS, D_H), jnp.float32),
            pltpu.VMEM((PASS, CH), jnp.int32),
            pltpu.VMEM((PASS, CH), jnp.int32),
            pltpu.VMEM((NBUF, CH, D_P), jnp.int32),
            pltpu.VMEM((NBUF, CH, D_H), jnp.float32),
        ] + [pltpu.SemaphoreType.DMA] * (2 * NBUF),
    )
    deg = pl.kernel(
        _sc_deg_body,
        out_type=out_t,
        mesh=mesh,
        compiler_params=params,
        scratch_types=[
            pltpu.VMEM_SHARED((A_ROWS, D_H), jnp.float32),
            pltpu.VMEM((K_DEG, CH), jnp.int32),
            pltpu.VMEM((CH, D_H), jnp.float32),
        ],
    )
    return agg, deg


BR = 1000
NB = N // BR


def _pack_bf16(v):
    lows = jnp.concatenate([v[:, 0:16], v[:, 32:48]], axis=1)
    highs = jnp.concatenate([v[:, 16:32], v[:, 48:64]], axis=1)
    lo = lax.bitcast_convert_type(lows.astype(jnp.bfloat16),
                                  jnp.uint16).astype(jnp.uint32)
    hi = lax.bitcast_convert_type(highs.astype(jnp.bfloat16),
                                  jnp.uint16).astype(jnp.uint32)
    return lax.bitcast_convert_type(lo | (hi << 16), jnp.int32)


def _tc_first_body(x_ref, degp_ref, w_ref, g_ref, hp_ref, dis_ref):
    deg = degp_ref[0] + degp_ref[1] + 1.0
    dis = lax.rsqrt(deg)
    g = jnp.dot(x_ref[...], w_ref[...], preferred_element_type=jnp.float32)
    g_ref[...] = g
    dis_ref[...] = dis
    hp_ref[...] = _pack_bf16(g * dis)


def _tc_layer_body(p_ref, g_ref, dis_ref, b_ref, w_ref, go_ref, hp_ref):
    dis = dis_ref[...]
    h = dis * (p_ref[0] + p_ref[1]) + dis * dis * g_ref[...] + b_ref[0:1, :]
    h = jnp.maximum(h, 0.0)
    g = jnp.dot(h, w_ref[...], preferred_element_type=jnp.float32)
    go_ref[...] = g
    hp_ref[...] = _pack_bf16(g * dis)


def _tc_pool_body(p_ref, g_ref, dis_ref, b_ref, batch_ref, out_ref, cnt_ref):
    i = pl.program_id(0)
    dis = dis_ref[...]
    h = dis * (p_ref[0] + p_ref[1]) + dis * dis * g_ref[...] + b_ref[0:1, :]
    h = jnp.maximum(h, 0.0)
    oh = (batch_ref[0, 0, :][None, :]
          == lax.broadcasted_iota(jnp.int32, (G, BR), 0)).astype(jnp.float32)
    part = jnp.dot(oh, h, preferred_element_type=jnp.float32)
    cpart = jnp.dot(oh, jnp.ones((BR, D_H), jnp.float32),
                    preferred_element_type=jnp.float32)

    @pl.when(i == 0)
    def _():
        out_ref[...] = jnp.zeros_like(out_ref)
        cnt_ref[...] = jnp.zeros_like(cnt_ref)

    out_ref[...] += part
    cnt_ref[...] += cpart

    @pl.when(i == NB - 1)
    def _():
        out_ref[...] = out_ref[...] / jnp.maximum(cnt_ref[...], 1.0)


_row_spec = pl.BlockSpec((BR, D_H), lambda i: (i, 0))
_degp_spec = pl.BlockSpec((NC, BR, D_H), lambda i: (0, i, 0))
_part_spec = _degp_spec
_b_spec = pl.BlockSpec((8, D_H), lambda i: (0, 0))
_w_spec = pl.BlockSpec((D_H, D_H), lambda i: (0, 0))

_hp_spec = pl.BlockSpec((BR, D_P), lambda i: (i, 0))
_hp_shape = jax.ShapeDtypeStruct((N, D_P), jnp.int32)

_tc_first = pl.pallas_call(
    _tc_first_body,
    grid=(NB,),
    in_specs=[
        pl.BlockSpec((BR, D_IN), lambda i: (i, 0)),
        _degp_spec,
        pl.BlockSpec((D_IN, D_H), lambda i: (0, 0)),
    ],
    out_specs=[_row_spec, _hp_spec, _row_spec],
    out_shape=[jax.ShapeDtypeStruct((N, D_H), jnp.float32), _hp_shape,
               jax.ShapeDtypeStruct((N, D_H), jnp.float32)],
)

_tc_layer = pl.pallas_call(
    _tc_layer_body,
    grid=(NB,),
    in_specs=[_part_spec, _row_spec, _row_spec, _b_spec, _w_spec],
    out_specs=[_row_spec, _hp_spec],
    out_shape=[jax.ShapeDtypeStruct((N, D_H), jnp.float32), _hp_shape],
)

_tc_pool = pl.pallas_call(
    _tc_pool_body,
    grid=(NB,),
    in_specs=[
        _part_spec, _row_spec, _row_spec, _b_spec,
        pl.BlockSpec((1, 1, BR), lambda i: (i, 0, 0)),
    ],
    out_specs=pl.BlockSpec((G, D_H), lambda i: (0, 0)),
    out_shape=jax.ShapeDtypeStruct((G, D_H), jnp.float32),
    scratch_shapes=[pltpu.VMEM((G, D_H), jnp.float32)],
)


@jax.jit
def kernel(x, edge_index, batch, W0, b0, W1, b1, W2, b2, W3, b3):
    src = edge_index[0]
    dst = edge_index[1]
    pad = E_PAD - E
    if pad:
        sink = N + jnp.arange(pad, dtype=jnp.int32) % (A_ROWS - N)
        src = jnp.concatenate([src, jnp.zeros((pad,), jnp.int32)])
        dst = jnp.concatenate([dst, sink])
    src2d = src.reshape(-1, CH)
    dst2d = dst.reshape(-1, CH)
    zeros_hbm = jnp.zeros((CH, D_H), jnp.float32)
    ones_hbm = jnp.ones((CH, D_H), jnp.float32)

    _sc_agg, _sc_deg = _sc_kernels()
    degp = _sc_deg(dst2d, ones_hbm, zeros_hbm)

    g, hp, dis = _tc_first(x, degp, W0)

    batch3 = batch.reshape(NB, 1, BR)
    bs = [jnp.tile(b.reshape(1, D_H), (8, 1)) for b in (b0, b1, b2, b3)]
    for b8, Wn in ((bs[0], W1), (bs[1], W2), (bs[2], W3)):
        parts = _sc_agg(hp, src2d, dst2d, zeros_hbm)
        g, hp = _tc_layer(parts, g, dis, b8, Wn)

    parts = _sc_agg(hp, src2d, dst2d, zeros_hbm)
    return _tc_pool(parts, g, dis, bs[3], batch3)


# speedup vs baseline: 1.0327x; 1.0327x over previous
import functools

import jax
import jax.numpy as jnp
from jax import lax
from jax.experimental import pallas as pl
from jax.experimental.pallas import tpu as pltpu
from jax.experimental.pallas import tpu_sc as plsc

N = 10000
E = 320000
D_IN = 128
D_H = 64
G = 64

NC = 2
NS = 16
CH = 128
NW = NC * NS
K_T = 80
NP = 1
PASS = K_T // NP
R_CHUNKS = NW * K_T
E_PAD = R_CHUNKS * CH
ROWS_T = 640
A_ROWS = NS * ROWS_T

NBUF = 4
LA = NBUF // 2
K_DEG = R_CHUNKS // NW
CVT_U = 4
D_P = D_H // 2


def _sc_agg_body(hp_hbm, src_hbm, dst_hbm, zeros_hbm, out_hbm,
                 acc, src_idx, dst_idx, rows16, rows, *sems):
    c = lax.axis_index("c")
    s = lax.axis_index("s")
    wid = c * NS + s
    row0 = s * ROWS_T

    pltpu.sync_copy(zeros_hbm, rows.at[0])
    for r in range(ROWS_T // CH):
        pltpu.sync_copy(rows.at[0], acc.at[pl.ds(row0 + r * CH, CH)])

    plsc.subcore_barrier()

    gsem = sems[:NBUF]
    ssem = sems[NBUF:]

    hi_mask = jnp.int32(-65536)

    def _convert(b):
        def crow(r, carry):
            for u in range(CVT_U):
                rr = r * CVT_U + u
                w1 = rows16[b, rr, 0:16]
                w2 = rows16[b, rr, 16:32]
                rows[b, rr, 0:16] = plsc.bitcast(w1 << 16, jnp.float32)
                rows[b, rr, 16:32] = plsc.bitcast(w1 & hi_mask, jnp.float32)
                rows[b, rr, 32:48] = plsc.bitcast(w2 << 16, jnp.float32)
                rows[b, rr, 48:64] = plsc.bitcast(w2 & hi_mask, jnp.float32)
            return carry

        lax.fori_loop(0, CH // CVT_U, crow, 0)

    def _step(j, b, swait, gfire):
        pltpu.make_async_copy(
            hp_hbm.at[src_idx.at[j]], rows16.at[b], gsem[b]).wait()
        _convert(b)
        pltpu.async_copy(rows.at[b], acc.at[dst_idx.at[j]], ssem[b],
                         add=True)
        b2 = (b + LA) % NBUF
        if swait:
            pltpu.make_async_copy(
                rows.at[b2], acc.at[dst_idx.at[j - LA]], ssem[b2]).wait()
        if gfire:
            pltpu.async_copy(hp_hbm.at[src_idx.at[j + LA]], rows16.at[b2],
                             gsem[b2])

    for p in range(NP):
        start = wid * K_T + p * PASS
        pltpu.sync_copy(src_hbm.at[pl.ds(start, PASS)], src_idx)
        pltpu.sync_copy(dst_hbm.at[pl.ds(start, PASS)], dst_idx)

        for b in range(LA):
            pltpu.async_copy(hp_hbm.at[src_idx.at[b]], rows16.at[b], gsem[b])

        for b in range(NBUF):
            _step(b, b, b >= LA, True)

        def step(i, carry):
            for b in range(NBUF):
                _step(i * NBUF + b, b, True, True)
            return carry

        lax.fori_loop(1, PASS // NBUF - 1, step, 0)

        for b in range(NBUF):
            _step(PASS - NBUF + b, b, True, b < NBUF - LA)

        for b in range(NBUF - LA, NBUF):
            pltpu.make_async_copy(
                rows.at[b], acc.at[dst_idx.at[PASS - NBUF + b]],
                ssem[b]).wait()

    plsc.subcore_barrier()

    for r in range(ROWS_T // CH):
        pltpu.sync_copy(acc.at[pl.ds(row0 + r * CH, CH)], rows.at[0])
        pltpu.sync_copy(rows.at[0], out_hbm.at[c, pl.ds(row0 + r * CH, CH)])


def _sc_deg_body(dst_hbm, ones_hbm, zeros_hbm, out_hbm,
                 acc, dst_idx, rows):
    c = lax.axis_index("c")
    s = lax.axis_index("s")
    wid = c * NS + s
    row0 = s * ROWS_T

    pltpu.sync_copy(zeros_hbm, rows)
    for r in range(ROWS_T // CH):
        pltpu.sync_copy(rows, acc.at[pl.ds(row0 + r * CH, CH)])
    pltpu.sync_copy(dst_hbm.at[pl.ds(wid * K_DEG, K_DEG)], dst_idx)
    pltpu.sync_copy(ones_hbm, rows)

    plsc.subcore_barrier()

    def body(j, carry):
        pltpu.sync_copy(rows, acc.at[dst_idx.at[j]], add=True)
        return carry

    lax.fori_loop(0, K_DEG, body, 0)

    plsc.subcore_barrier()

    for r in range(ROWS_T // CH):
        pltpu.sync_copy(acc.at[pl.ds(row0 + r * CH, CH)], rows)
        pltpu.sync_copy(rows, out_hbm.at[c, pl.ds(row0 + r * CH, CH)])


@functools.lru_cache(maxsize=1)
def _sc_kernels():
    mesh = plsc.VectorSubcoreMesh(
        core_axis_name="c", subcore_axis_name="s",
        num_cores=NC, num_subcores=NS,
    )
    out_t = jax.ShapeDtypeStruct((NC, A_ROWS, D_H), jnp.float32)
    params = pltpu.CompilerParams(use_tc_tiling_on_sc=False,
                                  needs_layout_passes=False)
    agg = pl.kernel(
        _sc_agg_body,
        out_type=out_t,
        mesh=mesh,
        compiler_params=params,
        scratch_types=[
            pltpu.VMEM_SHARED((A_ROWS, D_H), jnp.float32),
            pltpu.VMEM((PASS, CH), jnp.int32),
            pltpu.VMEM((PASS, CH), jnp.int32),
            pltpu.VMEM((NBUF, CH, D_P), jnp.int32),
            pltpu.VMEM((NBUF, CH, D_H), jnp.float32),
        ] + [pltpu.SemaphoreType.DMA] * (2 * NBUF),
    )
    deg = pl.kernel(
        _sc_deg_body,
        out_type=out_t,
        mesh=mesh,
        compiler_params=params,
        scratch_types=[
            pltpu.VMEM_SHARED((A_ROWS, D_H), jnp.float32),
            pltpu.VMEM((K_DEG, CH), jnp.int32),
            pltpu.VMEM((CH, D_H), jnp.float32),
        ],
    )
    return agg, deg


BR = 1000
NB = N // BR


def _pack_bf16(v):
    lows = jnp.concatenate([v[:, 0:16], v[:, 32:48]], axis=1)
    highs = jnp.concatenate([v[:, 16:32], v[:, 48:64]], axis=1)
    lo = lax.bitcast_convert_type(lows.astype(jnp.bfloat16),
                                  jnp.uint16).astype(jnp.uint32)
    hi = lax.bitcast_convert_type(highs.astype(jnp.bfloat16),
                                  jnp.uint16).astype(jnp.uint32)
    return lax.bitcast_convert_type(lo | (hi << 16), jnp.int32)


def _tc_first_body(x_ref, degp_ref, w_ref, g_ref, hp_ref, dis_ref):
    deg = degp_ref[0] + degp_ref[1] + 1.0
    dis = lax.rsqrt(deg)
    g = jnp.dot(x_ref[...], w_ref[...], preferred_element_type=jnp.float32)
    g_ref[...] = g
    dis_ref[...] = dis
    hp_ref[...] = _pack_bf16(g * dis)


def _tc_layer_body(p_ref, g_ref, dis_ref, b_ref, w_ref, go_ref, hp_ref):
    dis = dis_ref[...]
    h = dis * (p_ref[0] + p_ref[1]) + dis * dis * g_ref[...] + b_ref[0:1, :]
    h = jnp.maximum(h, 0.0)
    g = jnp.dot(h, w_ref[...], preferred_element_type=jnp.float32)
    go_ref[...] = g
    hp_ref[...] = _pack_bf16(g * dis)


def _tc_pool_body(p_ref, g_ref, dis_ref, b_ref, batch_ref, out_ref, cnt_ref):
    i = pl.program_id(0)
    dis = dis_ref[...]
    h = dis * (p_ref[0] + p_ref[1]) + dis * dis * g_ref[...] + b_ref[0:1, :]
    h = jnp.maximum(h, 0.0)
    oh = (batch_ref[0, 0, :][None, :]
          == lax.broadcasted_iota(jnp.int32, (G, BR), 0)).astype(jnp.float32)
    part = jnp.dot(oh, h, preferred_element_type=jnp.float32)
    cpart = jnp.dot(oh, jnp.ones((BR, D_H), jnp.float32),
                    preferred_element_type=jnp.float32)

    @pl.when(i == 0)
    def _():
        out_ref[...] = jnp.zeros_like(out_ref)
        cnt_ref[...] = jnp.zeros_like(cnt_ref)

    out_ref[...] += part
    cnt_ref[...] += cpart

    @pl.when(i == NB - 1)
    def _():
        out_ref[...] = out_ref[...] / jnp.maximum(cnt_ref[...], 1.0)


_row_spec = pl.BlockSpec((BR, D_H), lambda i: (i, 0))
_degp_spec = pl.BlockSpec((NC, BR, D_H), lambda i: (0, i, 0))
_part_spec = _degp_spec
_b_spec = pl.BlockSpec((8, D_H), lambda i: (0, 0))
_w_spec = pl.BlockSpec((D_H, D_H), lambda i: (0, 0))

_hp_spec = pl.BlockSpec((BR, D_P), lambda i: (i, 0))
_hp_shape = jax.ShapeDtypeStruct((N, D_P), jnp.int32)

_tc_first = pl.pallas_call(
    _tc_first_body,
    grid=(NB,),
    in_specs=[
        pl.BlockSpec((BR, D_IN), lambda i: (i, 0)),
        _degp_spec,
        pl.BlockSpec((D_IN, D_H), lambda i: (0, 0)),
    ],
    out_specs=[_row_spec, _hp_spec, _row_spec],
    out_shape=[jax.ShapeDtypeStruct((N, D_H), jnp.float32), _hp_shape,
               jax.ShapeDtypeStruct((N, D_H), jnp.float32)],
)

_tc_layer = pl.pallas_call(
    _tc_layer_body,
    grid=(NB,),
    in_specs=[_part_spec, _row_spec, _row_spec, _b_spec, _w_spec],
    out_specs=[_row_spec, _hp_spec],
    out_shape=[jax.ShapeDtypeStruct((N, D_H), jnp.float32), _hp_shape],
)

_tc_pool = pl.pallas_call(
    _tc_pool_body,
    grid=(NB,),
    in_specs=[
        _part_spec, _row_spec, _row_spec, _b_spec,
        pl.BlockSpec((1, 1, BR), lambda i: (i, 0, 0)),
    ],
    out_specs=pl.BlockSpec((G, D_H), lambda i: (0, 0)),
    out_shape=jax.ShapeDtypeStruct((G, D_H), jnp.float32),
    scratch_shapes=[pltpu.VMEM((G, D_H), jnp.float32)],
)


@jax.jit
def kernel(x, edge_index, batch, W0, b0, W1, b1, W2, b2, W3, b3):
    src = edge_index[0]
    dst = edge_index[1]
    pad = E_PAD - E
    if pad:
        sink = N + jnp.arange(pad, dtype=jnp.int32) % (A_ROWS - N)
        src = jnp.concatenate([src, jnp.zeros((pad,), jnp.int32)])
        dst = jnp.concatenate([dst, sink])
    src2d = src.reshape(-1, CH)
    dst2d = dst.reshape(-1, CH)
    zeros_hbm = jnp.zeros((CH, D_H), jnp.float32)
    ones_hbm = jnp.ones((CH, D_H), jnp.float32)

    _sc_agg, _sc_deg = _sc_kernels()
    degp = _sc_deg(dst2d, ones_hbm, zeros_hbm)

    g, hp, dis = _tc_first(x, degp, W0)

    batch3 = batch.reshape(NB, 1, BR)
    bs = [jnp.tile(b.reshape(1, D_H), (8, 1)) for b in (b0, b1, b2, b3)]
    for b8, Wn in ((bs[0], W1), (bs[1], W2), (bs[2], W3)):
        parts = _sc_agg(hp, src2d, dst2d, zeros_hbm)
        g, hp = _tc_layer(parts, g, dis, b8, Wn)

    parts = _sc_agg(hp, src2d, dst2d, zeros_hbm)
    return _tc_pool(parts, g, dis, bs[3], batch3)
